# baseline (speedup 1.0000x reference)
import functools

import jax
import jax.numpy as jnp
from jax import lax
from jax.experimental import pallas as pl
from jax.experimental.pallas import tpu as pltpu
from jax.experimental.pallas import tpu_sc as plsc

D = 128
HOUR_SIZE = 72
MINUTE_SIZE = 60
BSTRIPE = 128
LGRP = 2


def _csum_body(hour_ref, minute_ref, out_ref):
    h = hour_ref[...]
    m = minute_ref[...]
    out_ref[...] = (h[:, None, :] + m[None, :, :]).reshape(-1, D)


def _build_csum(hour_table, minute_table):
    return pl.pallas_call(
        _csum_body,
        out_shape=jax.ShapeDtypeStruct((HOUR_SIZE * MINUTE_SIZE, D), jnp.float32),
    )(hour_table, minute_table)


def _make_sc_gather(b, l):
    n_tokens = b * l
    nbt = b // BSTRIPE
    info = plsc.get_sparse_core_info()
    nc, ns = info.num_cores, info.num_subcores
    nw = nc * ns
    n_groups = (l // LGRP) * nbt
    per_w = n_groups // nw
    assert per_w * nw == n_groups and per_w % 2 == 0
    mesh = plsc.VectorSubcoreMesh(core_axis_name="c", subcore_axis_name="s")

    @functools.partial(
        pl.kernel,
        out_type=jax.ShapeDtypeStruct((n_tokens, D), jnp.float32),
        mesh=mesh,
        compiler_params=pltpu.CompilerParams(needs_layout_passes=False),
        scratch_types=[
            pltpu.VMEM((LGRP * 2 * BSTRIPE,), jnp.int32),
            pltpu.VMEM((LGRP * 2 * BSTRIPE,), jnp.int32),
            pltpu.VMEM((BSTRIPE,), jnp.int32),
            pltpu.VMEM((BSTRIPE,), jnp.int32),
            pltpu.VMEM((BSTRIPE,), jnp.int32),
            pltpu.VMEM((BSTRIPE,), jnp.int32),
            pltpu.VMEM((BSTRIPE,), jnp.int32),
            pltpu.VMEM((BSTRIPE,), jnp.int32),
            pltpu.VMEM((BSTRIPE,), jnp.int32),
            pltpu.VMEM((BSTRIPE,), jnp.int32),
            pltpu.VMEM((2, LGRP * BSTRIPE, D), jnp.float32),
            pltpu.VMEM_SHARED((HOUR_SIZE * MINUTE_SIZE, D), jnp.float32),
            pltpu.SemaphoreType.DMA((2,)),
            pltpu.SemaphoreType.DMA((2,)),
            pltpu.SemaphoreType.DMA((2,)),
        ],
    )
    def sc_kernel(x_hbm, csum_hbm, out_hbm, xbuf0, xbuf1,
                  idx00, idx01, idx10, idx11,
                  oidx00, oidx01, oidx10, oidx11,
                  staged, shared_csum, sem_x, sem_g, sem_o):
        wid = lax.axis_index("s") * nc + lax.axis_index("c")
        w_base = wid * per_w
        lanes = lax.iota(jnp.int32, 16)
        xbufs = (xbuf0, xbuf1)
        idxs = ((idx00, idx01), (idx10, idx11))
        oidxs = ((oidx00, oidx01), (oidx10, oidx11))

        def issue_x(g, slot):
            gg = w_base + g
            l2 = gg // nbt
            bt = gg % nbt
            for ls in range(LGRP):
                pltpu.async_copy(
                    x_hbm.at[pl.ds(((l2 * LGRP + ls) * nbt + bt) * 2 * BSTRIPE,
                                   2 * BSTRIPE)],
                    xbufs[slot].at[pl.ds(ls * 2 * BSTRIPE, 2 * BSTRIPE)],
                    sem_x.at[slot])

        def wait_x(slot):
            for _ in range(LGRP):
                pltpu.make_async_copy(
                    x_hbm.at[pl.ds(0, 2 * BSTRIPE)],
                    xbufs[slot].at[pl.ds(0, 2 * BSTRIPE)],
                    sem_x.at[slot]).wait()

        def wait_o(slot):
            for ls in range(LGRP):
                pltpu.make_async_copy(
                    staged.at[slot, pl.ds(ls * BSTRIPE, BSTRIPE)],
                    out_hbm.at[pl.ds(0, BSTRIPE)],
                    sem_o.at[slot]).wait()

        def chunk(g, slot):
            gg = w_base + g
            l2 = gg // nbt
            bt = gg % nbt
            l0 = l2 * LGRP
            xb = xbufs[slot]
            wait_x(slot)
            rowc = bt * (BSTRIPE * l) + l0
            for ls in range(LGRP):
                for j in range(BSTRIPE // 16):
                    hv = xb[pl.ds(ls * 2 * BSTRIPE + j * 16, 16)]
                    mv = xb[pl.ds(ls * 2 * BSTRIPE + BSTRIPE + j * 16, 16)]
                    plsc.store_scatter(
                        idxs[slot][j // 4],
                        [2 * lanes + (32 * (j % 4) + ls)],
                        hv * MINUTE_SIZE + mv)
            for k in range(2):
                for j in range(BSTRIPE // 16):
                    oidxs[slot][k][pl.ds(j * 16, 16)] = (
                        (rowc + l * (64 * k) + l * ((j * 16 + lanes) >> 1))
                        + (lanes & 1))
            issue_x(jnp.minimum(g + 2, per_w - 1), slot)

            @pl.when(g >= 2)
            def _():
                wait_o(slot)

            for ls in range(LGRP):
                pltpu.async_copy(
                    shared_csum.at[idxs[slot][ls]],
                    staged.at[slot, pl.ds(ls * BSTRIPE, BSTRIPE)],
                    sem_g.at[slot])
            for ls in range(LGRP):
                pltpu.make_async_copy(
                    shared_csum.at[idxs[slot][ls]],
                    staged.at[slot, pl.ds(ls * BSTRIPE, BSTRIPE)],
                    sem_g.at[slot]).wait()
            for ls in range(LGRP):
                pltpu.async_copy(
                    staged.at[slot, pl.ds(ls * BSTRIPE, BSTRIPE)],
                    out_hbm.at[oidxs[slot][ls]],
                    sem_o.at[slot])

        issue_x(0, 0)
        issue_x(1, 1)

        @pl.when(lax.axis_index("s") == 0)
        def _():
            pltpu.sync_copy(csum_hbm, shared_csum)

        plsc.subcore_barrier()

        def outer(oo, _):
            chunk(2 * oo, 0)
            chunk(2 * oo + 1, 1)
            return 0

        lax.fori_loop(0, per_w // 2, outer, 0)
        for slot in (0, 1):
            wait_o(slot)
            wait_x(slot)

    return sc_kernel


def kernel(x, minute_table, hour_table):
    b, l, _ = x.shape
    nbt = b // BSTRIPE
    xr = x.astype(jnp.int32)
    xt = (xr.transpose(1, 0, 2)
            .reshape(l, nbt, BSTRIPE, 2)
            .transpose(0, 1, 3, 2)
            .reshape(-1))
    csum = _build_csum(hour_table, minute_table)
    out = _make_sc_gather(b, l)(xt, csum)
    return out.reshape(b, l, D)


# speedup vs baseline: 1.0268x; 1.0268x over previous
import functools

import jax
import jax.numpy as jnp
from jax import lax
from jax.experimental import pallas as pl
from jax.experimental.pallas import tpu as pltpu
from jax.experimental.pallas import tpu_sc as plsc

D = 128
HOUR_SIZE = 72
MINUTE_SIZE = 60
BSTRIPE = 128
LGRP = 2


def _csum_body(hour_ref, minute_ref, out_ref):
    h = hour_ref[...]
    m = minute_ref[...]
    out_ref[...] = (h[:, None, :] + m[None, :, :]).reshape(-1, D)


def _build_csum(hour_table, minute_table):
    return pl.pallas_call(
        _csum_body,
        out_shape=jax.ShapeDtypeStruct((HOUR_SIZE * MINUTE_SIZE, D), jnp.float32),
    )(hour_table, minute_table)


def _make_sc_gather(b, l):
    n_tokens = b * l
    nbt = b // BSTRIPE
    info = plsc.get_sparse_core_info()
    nc, ns = info.num_cores, info.num_subcores
    nw = nc * ns
    n_groups = (l // LGRP) * nbt
    per_w = n_groups // nw
    assert per_w * nw == n_groups and per_w % 2 == 0
    mesh = plsc.VectorSubcoreMesh(core_axis_name="c", subcore_axis_name="s")

    @functools.partial(
        pl.kernel,
        out_type=jax.ShapeDtypeStruct((n_tokens, D), jnp.float32),
        mesh=mesh,
        compiler_params=pltpu.CompilerParams(needs_layout_passes=False),
        scratch_types=[
            pltpu.VMEM((LGRP * 2 * BSTRIPE,), jnp.int32),
            pltpu.VMEM((LGRP * 2 * BSTRIPE,), jnp.int32),
            pltpu.VMEM((BSTRIPE,), jnp.int32),
            pltpu.VMEM((BSTRIPE,), jnp.int32),
            pltpu.VMEM((BSTRIPE,), jnp.int32),
            pltpu.VMEM((BSTRIPE,), jnp.int32),
            pltpu.VMEM((BSTRIPE,), jnp.int32),
            pltpu.VMEM((BSTRIPE,), jnp.int32),
            pltpu.VMEM((BSTRIPE,), jnp.int32),
            pltpu.VMEM((BSTRIPE,), jnp.int32),
            pltpu.VMEM((2, LGRP * BSTRIPE, D), jnp.float32),
            pltpu.VMEM_SHARED((HOUR_SIZE * MINUTE_SIZE, D), jnp.float32),
            pltpu.SemaphoreType.DMA((2,)),
            pltpu.SemaphoreType.DMA((2,)),
            pltpu.SemaphoreType.DMA((2,)),
        ],
    )
    def sc_kernel(x_hbm, csum_hbm, out_hbm, xbuf0, xbuf1,
                  idx00, idx01, idx10, idx11,
                  oidx00, oidx01, oidx10, oidx11,
                  staged, shared_csum, sem_x, sem_g, sem_o):
        wid = lax.axis_index("s") * nc + lax.axis_index("c")
        w_base = wid * per_w
        lanes = lax.iota(jnp.int32, 16)
        xbufs = (xbuf0, xbuf1)
        idxs = ((idx00, idx01), (idx10, idx11))
        oidxs = ((oidx00, oidx01), (oidx10, oidx11))

        def issue_x(g, slot):
            gg = w_base + g
            l2 = gg // nbt
            bt = gg % nbt
            for ls in range(LGRP):
                pltpu.async_copy(
                    x_hbm.at[pl.ds(((l2 * LGRP + ls) * nbt + bt) * 2 * BSTRIPE,
                                   2 * BSTRIPE)],
                    xbufs[slot].at[pl.ds(ls * 2 * BSTRIPE, 2 * BSTRIPE)],
                    sem_x.at[slot])

        def wait_x(slot):
            for _ in range(LGRP):
                pltpu.make_async_copy(
                    x_hbm.at[pl.ds(0, 2 * BSTRIPE)],
                    xbufs[slot].at[pl.ds(0, 2 * BSTRIPE)],
                    sem_x.at[slot]).wait()

        def wait_o(slot):
            for ls in range(LGRP):
                pltpu.make_async_copy(
                    staged.at[slot, pl.ds(ls * BSTRIPE, BSTRIPE)],
                    out_hbm.at[pl.ds(0, BSTRIPE)],
                    sem_o.at[slot]).wait()

        def chunk(g, slot):
            gg = w_base + g
            l2 = gg // nbt
            bt = gg % nbt
            l0 = l2 * LGRP
            xb = xbufs[slot]
            wait_x(slot)
            rowc = bt * (BSTRIPE * l) + l0
            for ls in range(LGRP):
                for k in range(BSTRIPE // 16):
                    hv = xb[pl.ds(ls * 2 * BSTRIPE + k * 16, 16)]
                    mv = xb[pl.ds(ls * 2 * BSTRIPE + BSTRIPE + k * 16, 16)]
                    idxs[slot][ls][pl.ds(k * 16, 16)] = hv * MINUTE_SIZE + mv
                    oidxs[slot][ls][pl.ds(k * 16, 16)] = (
                        (rowc + ls + l * k * 16) + l * lanes)
            issue_x(jnp.minimum(g + 2, per_w - 1), slot)

            @pl.when(g >= 2)
            def _():
                wait_o(slot)

            for ls in range(LGRP):
                pltpu.async_copy(
                    shared_csum.at[idxs[slot][ls]],
                    staged.at[slot, pl.ds(ls * BSTRIPE, BSTRIPE)],
                    sem_g.at[slot])
            for ls in range(LGRP):
                pltpu.make_async_copy(
                    shared_csum.at[idxs[slot][ls]],
                    staged.at[slot, pl.ds(ls * BSTRIPE, BSTRIPE)],
                    sem_g.at[slot]).wait()
            for ls in range(LGRP):
                pltpu.async_copy(
                    staged.at[slot, pl.ds(ls * BSTRIPE, BSTRIPE)],
                    out_hbm.at[oidxs[slot][ls]],
                    sem_o.at[slot])

        issue_x(0, 0)
        issue_x(1, 1)

        @pl.when(lax.axis_index("s") == 0)
        def _():
            pltpu.sync_copy(csum_hbm, shared_csum)

        plsc.subcore_barrier()

        def outer(oo, _):
            chunk(2 * oo, 0)
            chunk(2 * oo + 1, 1)
            return 0

        lax.fori_loop(0, per_w // 2, outer, 0)
        for slot in (0, 1):
            wait_o(slot)
            wait_x(slot)

    return sc_kernel


def kernel(x, minute_table, hour_table):
    b, l, _ = x.shape
    nbt = b // BSTRIPE
    xr = x.astype(jnp.int32)
    xt = (xr.transpose(1, 0, 2)
            .reshape(l, nbt, BSTRIPE, 2)
            .transpose(0, 1, 3, 2)
            .reshape(-1))
    csum = _build_csum(hour_table, minute_table)
    out = _make_sc_gather(b, l)(xt, csum)
    return out.reshape(b, l, D)
